# Initial kernel scaffold; baseline (speedup 1.0000x reference)
import jax
import jax.numpy as jnp
from jax.experimental import pallas as pl


def kernel(output, target):
    raise NotImplementedError("write your pallas kernel here")


# speedup vs baseline: 2.3359x; 2.3359x over previous
import functools

import jax
import jax.numpy as jnp
from jax import lax
from jax.experimental import pallas as pl
from jax.experimental.pallas import tpu as pltpu
from jax.experimental.pallas import tpu_sc as plsc

L = 16
NC = 2
NS = 16
NW = NC * NS
UNROLL = 8


def _insert(m1, m2, m3, x):
    n1 = jnp.maximum(m1, x)
    s1 = jnp.minimum(m1, x)
    n2 = jnp.maximum(m2, s1)
    s2 = jnp.minimum(m2, s1)
    n3 = jnp.maximum(m3, s2)
    return n1, n2, n3


def _remove_first(m1, m2, m3, t):
    eq = m1 == t
    cs = jnp.cumsum(eq.astype(jnp.int32))
    oh = jnp.logical_and(eq, cs == 1)
    return jnp.where(oh, m2, m1), jnp.where(oh, m3, m2)


def _top3(m1, m2, m3):
    t1 = jnp.max(m1)
    m1, m2 = _remove_first(m1, m2, m3, t1)
    t2 = jnp.max(m1)
    m1, _ = _remove_first(m1, m2, m3, t2)
    t3 = jnp.max(m1)
    full = lambda t: jnp.full((L,), t, jnp.float32)
    return full(t1), full(t2), full(t3)


def kernel(output, target):
    R, N = output.shape
    rows_per = R // NW
    steps = N // (L * UNROLL)

    mesh = plsc.VectorSubcoreMesh(core_axis_name="c", subcore_axis_name="s")

    @functools.partial(
        pl.kernel,
        mesh=mesh,
        out_type=jax.ShapeDtypeStruct((NW, L), jnp.float32),
        scratch_types=[
            pltpu.VMEM((N,), jnp.float32),
            pltpu.VMEM((N,), jnp.float32),
            pltpu.VMEM((L,), jnp.float32),
        ],
        compiler_params=pltpu.CompilerParams(needs_layout_passes=False),
    )
    def sc_loss(out_hbm, tgt_hbm, part_hbm, buf_o, buf_t, buf_p):
        wid = lax.axis_index("s") * NC + lax.axis_index("c")
        neg = jnp.full((L,), -jnp.inf, jnp.float32)
        acc = jnp.zeros((L,), jnp.float32)
        for r in range(rows_per):
            row = wid * rows_per + r
            pltpu.sync_copy(out_hbm.at[row], buf_o)
            pltpu.sync_copy(tgt_hbm.at[row], buf_t)

            def step(i, carry):
                o1, o2, o3, t1, t2, t3 = carry
                for u in range(UNROLL):
                    base = (i * UNROLL + u) * L
                    x = buf_o[pl.ds(base, L)]
                    y = buf_t[pl.ds(base, L)]
                    o1, o2, o3 = _insert(o1, o2, o3, x)
                    t1, t2, t3 = _insert(t1, t2, t3, y)
                return o1, o2, o3, t1, t2, t3

            o1, o2, o3, t1, t2, t3 = lax.fori_loop(
                0, steps, step, (neg, neg, neg, neg, neg, neg))
            a1, a2, a3 = _top3(o1, o2, o3)
            b1, b2, b3 = _top3(t1, t2, t3)
            d1 = a1 - b1
            d2 = a2 - b2
            d3 = a3 - b3
            acc = acc + 9.0 * d1 * d1 + 4.0 * d2 * d2 + d3 * d3
        buf_p[...] = acc
        pltpu.sync_copy(buf_p, part_hbm.at[wid])

    parts = sc_loss(output, target)
    return jnp.sum(parts[:, 0]) / jnp.float32(R * 3)


# speedup vs baseline: 2.9937x; 1.2816x over previous
import functools

import jax
import jax.numpy as jnp
from jax import lax
from jax.experimental import pallas as pl
from jax.experimental.pallas import tpu as pltpu
from jax.experimental.pallas import tpu_sc as plsc

L = 16
NC = 2
NS = 16
NW = NC * NS
UNROLL = 8
HALVES = 2


def _insert(m1, m2, m3, x):
    n1 = jnp.maximum(m1, x)
    s1 = jnp.minimum(m1, x)
    n2 = jnp.maximum(m2, s1)
    s2 = jnp.minimum(m2, s1)
    n3 = jnp.maximum(m3, s2)
    return n1, n2, n3


def _remove_first(m1, m2, m3, t):
    eq = m1 == t
    cs = jnp.cumsum(eq.astype(jnp.int32))
    oh = jnp.logical_and(eq, cs == 1)
    return jnp.where(oh, m2, m1), jnp.where(oh, m3, m2)


def _top3(m1, m2, m3):
    t1 = jnp.max(m1)
    m1, m2 = _remove_first(m1, m2, m3, t1)
    t2 = jnp.max(m1)
    m1, _ = _remove_first(m1, m2, m3, t2)
    t3 = jnp.max(m1)
    full = lambda t: jnp.full((L,), t, jnp.float32)
    return full(t1), full(t2), full(t3)


def kernel(output, target):
    R, N = output.shape
    rows_per = R // NW
    ch = N // HALVES
    steps = ch // (L * UNROLL)
    nchunks = rows_per * HALVES

    mesh = plsc.VectorSubcoreMesh(core_axis_name="c", subcore_axis_name="s")

    @functools.partial(
        pl.kernel,
        mesh=mesh,
        out_type=jax.ShapeDtypeStruct((NW, L), jnp.float32),
        scratch_types=[
            pltpu.VMEM((ch,), jnp.float32),
            pltpu.VMEM((ch,), jnp.float32),
            pltpu.VMEM((ch,), jnp.float32),
            pltpu.VMEM((ch,), jnp.float32),
            pltpu.VMEM((L,), jnp.float32),
            pltpu.SemaphoreType.DMA,
            pltpu.SemaphoreType.DMA,
        ],
        compiler_params=pltpu.CompilerParams(needs_layout_passes=False),
    )
    def sc_loss(out_hbm, tgt_hbm, part_hbm, bo0, bt0, bo1, bt1, buf_p, s0, s1):
        wid = lax.axis_index("s") * NC + lax.axis_index("c")
        slots = [(bo0, bt0, s0), (bo1, bt1, s1)]
        neg = jnp.full((L,), -jnp.inf, jnp.float32)
        acc = jnp.zeros((L,), jnp.float32)

        def start(s):
            row = wid * rows_per + s // HALVES
            off = (s % HALVES) * ch
            bo, bt, sem = slots[s % 2]
            h1 = pltpu.async_copy(out_hbm.at[row, pl.ds(off, ch)], bo, sem)
            h2 = pltpu.async_copy(tgt_hbm.at[row, pl.ds(off, ch)], bt, sem)
            return h1, h2

        inflight = {0: start(0)}
        carry = None
        for s in range(nchunks):
            if s + 1 < nchunks:
                inflight[s + 1] = start(s + 1)
            h1, h2 = inflight.pop(s)
            h1.wait()
            h2.wait()
            bo, bt, _ = slots[s % 2]
            if s % HALVES == 0:
                carry = (neg, neg, neg, neg, neg, neg)

            def step(i, c, bo=bo, bt=bt):
                o1, o2, o3, t1, t2, t3 = c
                for u in range(UNROLL):
                    base = (i * UNROLL + u) * L
                    x = bo[pl.ds(base, L)]
                    y = bt[pl.ds(base, L)]
                    o1, o2, o3 = _insert(o1, o2, o3, x)
                    t1, t2, t3 = _insert(t1, t2, t3, y)
                return o1, o2, o3, t1, t2, t3

            carry = lax.fori_loop(0, steps, step, carry)
            if s % HALVES == HALVES - 1:
                o1, o2, o3, t1, t2, t3 = carry
                a1, a2, a3 = _top3(o1, o2, o3)
                b1, b2, b3 = _top3(t1, t2, t3)
                d1 = a1 - b1
                d2 = a2 - b2
                d3 = a3 - b3
                acc = acc + 9.0 * d1 * d1 + 4.0 * d2 * d2 + d3 * d3
        buf_p[...] = acc
        pltpu.sync_copy(buf_p, part_hbm.at[wid])

    parts = sc_loss(output, target)
    return jnp.sum(parts[:, 0]) / jnp.float32(R * 3)


# speedup vs baseline: 3.0817x; 1.0294x over previous
import functools

import jax
import jax.numpy as jnp
from jax import lax
from jax.experimental import pallas as pl
from jax.experimental.pallas import tpu as pltpu
from jax.experimental.pallas import tpu_sc as plsc

L = 16
NC = 2
NS = 16
NW = NC * NS
UNROLL = 8
HALVES = 2


def _insert(m1, m2, m3, x):
    n1 = jnp.maximum(m1, x)
    s1 = jnp.minimum(m1, x)
    n2 = jnp.maximum(m2, s1)
    s2 = jnp.minimum(m2, s1)
    n3 = jnp.maximum(m3, s2)
    return n1, n2, n3


def _remove_first(m1, m2, m3, t):
    eq = m1 == t
    cs = jnp.cumsum(eq.astype(jnp.int32))
    oh = jnp.logical_and(eq, cs == 1)
    return jnp.where(oh, m2, m1), jnp.where(oh, m3, m2)


def _top3(m1, m2, m3):
    t1 = jnp.max(m1)
    m1, m2 = _remove_first(m1, m2, m3, t1)
    t2 = jnp.max(m1)
    m1, _ = _remove_first(m1, m2, m3, t2)
    t3 = jnp.max(m1)
    full = lambda t: jnp.full((L,), t, jnp.float32)
    return full(t1), full(t2), full(t3)


def kernel(output, target):
    R, N = output.shape
    rows_per = R // NW
    ch = N // HALVES
    steps = ch // (L * UNROLL)
    nchunks = rows_per * HALVES

    mesh = plsc.VectorSubcoreMesh(core_axis_name="c", subcore_axis_name="s")

    @functools.partial(
        pl.kernel,
        mesh=mesh,
        out_type=jax.ShapeDtypeStruct((NW, L), jnp.float32),
        scratch_types=[
            pltpu.VMEM((ch,), jnp.float32),
            pltpu.VMEM((ch,), jnp.float32),
            pltpu.VMEM((ch,), jnp.float32),
            pltpu.VMEM((ch,), jnp.float32),
            pltpu.VMEM((L,), jnp.float32),
            pltpu.SemaphoreType.DMA,
            pltpu.SemaphoreType.DMA,
        ],
        compiler_params=pltpu.CompilerParams(needs_layout_passes=False),
    )
    def sc_loss(out_hbm, tgt_hbm, part_hbm, bo0, bt0, bo1, bt1, buf_p, s0, s1):
        wid = lax.axis_index("s") * NC + lax.axis_index("c")
        slots = [(bo0, bt0, s0), (bo1, bt1, s1)]
        neg = jnp.full((L,), -jnp.inf, jnp.float32)
        row0 = wid * rows_per

        def copies(row, h):
            bo, bt, sem = slots[h]
            off = h * ch
            return (pltpu.make_async_copy(out_hbm.at[row, pl.ds(off, ch)], bo, sem),
                    pltpu.make_async_copy(tgt_hbm.at[row, pl.ds(off, ch)], bt, sem))

        for h in range(HALVES):
            for c in copies(row0, h):
                c.start()

        def row_body(r, acc):
            row = row0 + r
            carry = (neg, neg, neg, neg, neg, neg)
            for h in range(HALVES):
                for c in copies(row, h):
                    c.wait()
                bo, bt, _ = slots[h]

                def step(i, c, bo=bo, bt=bt):
                    o1, o2, o3, t1, t2, t3 = c
                    for u in range(UNROLL):
                        base = (i * UNROLL + u) * L
                        x = bo[pl.ds(base, L)]
                        y = bt[pl.ds(base, L)]
                        o1, o2, o3 = _insert(o1, o2, o3, x)
                        t1, t2, t3 = _insert(t1, t2, t3, y)
                    return o1, o2, o3, t1, t2, t3

                carry = lax.fori_loop(0, steps, step, carry)

                @pl.when(r + 1 < rows_per)
                def _():
                    for c in copies(row + 1, h):
                        c.start()

            o1, o2, o3, t1, t2, t3 = carry
            a1, a2, a3 = _top3(o1, o2, o3)
            b1, b2, b3 = _top3(t1, t2, t3)
            d1 = a1 - b1
            d2 = a2 - b2
            d3 = a3 - b3
            return acc + 9.0 * d1 * d1 + 4.0 * d2 * d2 + d3 * d3

        acc = lax.fori_loop(0, rows_per, row_body, jnp.zeros((L,), jnp.float32))
        buf_p[...] = acc
        pltpu.sync_copy(buf_p, part_hbm.at[wid])

    parts = sc_loss(output, target)
    return jnp.sum(parts[:, 0]) / jnp.float32(R * 3)
